# initial kernel scaffold (unmeasured)
import jax
import jax.numpy as jnp
from jax import lax
from jax.experimental import pallas as pl
from jax.experimental.pallas import tpu as pltpu

N_DEV = 4


def kernel(x, w_mat):
    m_glob, k_shard = x.shape
    k_glob, n = w_mat.shape
    m_per = m_glob // N_DEV

    x = x.astype(jnp.bfloat16)
    w_mat = w_mat.astype(jnp.bfloat16)

    def body(x_ref, w_ref, out_ref, recv_ref, acc_ref, amax_src, amax_recv,
             send_sems, recv_sems, a_send_sems, a_recv_sems):
        me = lax.axis_index("i")

        barrier_sem = pltpu.get_barrier_semaphore()
        for h in range(1, N_DEV):
            peer = lax.rem(me + h, N_DEV)
            pl.semaphore_signal(
                barrier_sem, inc=1,
                device_id=(peer,), device_id_type=pl.DeviceIdType.MESH,
            )
        pl.semaphore_wait(barrier_sem, N_DEV - 1)

        rdmas = []
        for h in range(1, N_DEV):
            t = lax.rem(me + h, N_DEV)
            rdma = pltpu.make_async_remote_copy(
                src_ref=x_ref.at[pl.ds(t * m_per, m_per), :],
                dst_ref=recv_ref.at[h - 1],
                send_sem=send_sems.at[h - 1],
                recv_sem=recv_sems.at[h - 1],
                device_id=(t,),
                device_id_type=pl.DeviceIdType.MESH,
            )
            rdma.start()
            rdmas.append(rdma)

        acc_ref[...] = jnp.dot(
            x_ref[pl.ds(me * m_per, m_per), :],
            w_ref[pl.ds(me * k_shard, k_shard), :],
            preferred_element_type=jnp.float32,
        )

        for h in (1, 3, 2):
            rdmas[h - 1].wait_recv()
            s = lax.rem(me + N_DEV - h, N_DEV)
            acc_ref[...] += jnp.dot(
                recv_ref[h - 1],
                w_ref[pl.ds(s * k_shard, k_shard), :],
                preferred_element_type=jnp.float32,
            )

        for h in range(1, N_DEV):
            rdmas[h - 1].wait_send()

        y = jnp.maximum(acc_ref[...], 0.0)
        out_ref[...] = y
        la = jnp.max(y)
        amax_src[...] = jnp.full((8, 128), la, jnp.float32)

        a_rdmas = []
        for h in range(1, N_DEV):
            t = lax.rem(me + h, N_DEV)
            r = pltpu.make_async_remote_copy(
                src_ref=amax_src,
                dst_ref=amax_recv.at[h - 1],
                send_sem=a_send_sems.at[h - 1],
                recv_sem=a_recv_sems.at[h - 1],
                device_id=(t,),
                device_id_type=pl.DeviceIdType.MESH,
            )
            r.start()
            a_rdmas.append(r)
        for r in a_rdmas:
            r.wait()

        g_amax = jnp.maximum(la, jnp.max(amax_recv[...]))
        scale = g_amax / 448.0
        q = (out_ref[...] / scale).astype(jnp.float8_e4m3fn)
        out_ref[...] = q.astype(jnp.float32) * scale

    return pl.pallas_call(
        body,
        out_shape=jax.ShapeDtypeStruct((m_per, n), jnp.float32),
        in_specs=[
            pl.BlockSpec(memory_space=pltpu.VMEM),
            pl.BlockSpec(memory_space=pltpu.VMEM),
        ],
        out_specs=pl.BlockSpec(memory_space=pltpu.VMEM),
        scratch_shapes=[
            pltpu.VMEM((N_DEV - 1, m_per, k_shard), jnp.bfloat16),
            pltpu.VMEM((m_per, n), jnp.float32),
            pltpu.VMEM((8, 128), jnp.float32),
            pltpu.VMEM((N_DEV - 1, 8, 128), jnp.float32),
            pltpu.SemaphoreType.DMA((N_DEV - 1,)),
            pltpu.SemaphoreType.DMA((N_DEV - 1,)),
            pltpu.SemaphoreType.DMA((N_DEV - 1,)),
            pltpu.SemaphoreType.DMA((N_DEV - 1,)),
        ],
        compiler_params=pltpu.CompilerParams(collective_id=0),
    )(x, w_mat)


# baseline (device time: 111068 ns/iter reference)
import jax
import jax.numpy as jnp
from jax import lax
from jax.experimental import pallas as pl
from jax.experimental.pallas import tpu as pltpu

N_DEV = 4


def kernel(x, w_mat):
    m_glob, k_shard = x.shape
    k_glob, n = w_mat.shape
    m_per = m_glob // N_DEV

    x = x.astype(jnp.bfloat16)
    w_mat = w_mat.astype(jnp.bfloat16)

    def body(x_ref, w_ref, out_ref, recv_ref, acc_ref, amax_src, amax_recv,
             send_sems, recv_sems, a_send_sems, a_recv_sems):
        me = lax.axis_index("i")

        barrier_sem = pltpu.get_barrier_semaphore()
        for h in range(1, N_DEV):
            peer = lax.rem(me + h, N_DEV)
            pl.semaphore_signal(
                barrier_sem, inc=1,
                device_id=(peer,), device_id_type=pl.DeviceIdType.MESH,
            )
        pl.semaphore_wait(barrier_sem, N_DEV - 1)

        rdmas = []
        for h in range(1, N_DEV):
            t = lax.rem(me + h, N_DEV)
            rdma = pltpu.make_async_remote_copy(
                src_ref=x_ref.at[pl.ds(t * m_per, m_per), :],
                dst_ref=recv_ref.at[h - 1],
                send_sem=send_sems.at[h - 1],
                recv_sem=recv_sems.at[h - 1],
                device_id=(t,),
                device_id_type=pl.DeviceIdType.MESH,
            )
            rdma.start()
            rdmas.append(rdma)

        acc_ref[...] = jnp.dot(
            x_ref[pl.ds(me * m_per, m_per), :],
            w_ref[pl.ds(me * k_shard, k_shard), :],
            preferred_element_type=jnp.float32,
        )

        for h in (1, 3, 2):
            rdmas[h - 1].wait_recv()
            s = lax.rem(me + N_DEV - h, N_DEV)
            acc_ref[...] += jnp.dot(
                recv_ref[h - 1],
                w_ref[pl.ds(s * k_shard, k_shard), :],
                preferred_element_type=jnp.float32,
            )

        for h in range(1, N_DEV):
            rdmas[h - 1].wait_send()

        y = jnp.maximum(acc_ref[...], 0.0)
        out_ref[...] = y
        la = jnp.max(y)
        amax_src[...] = jnp.full((8, 128), la, jnp.float32)

        a_rdmas = []
        for h in range(1, N_DEV):
            t = lax.rem(me + h, N_DEV)
            r = pltpu.make_async_remote_copy(
                src_ref=amax_src,
                dst_ref=amax_recv.at[h - 1],
                send_sem=a_send_sems.at[h - 1],
                recv_sem=a_recv_sems.at[h - 1],
                device_id=(t,),
                device_id_type=pl.DeviceIdType.MESH,
            )
            r.start()
            a_rdmas.append(r)
        for r in a_rdmas:
            r.wait()

        g_amax = jnp.maximum(la, jnp.max(amax_recv[...]))
        scale = g_amax / 448.0
        q = (out_ref[...] / scale).astype(jnp.float8_e4m3fn)
        out_ref[...] = q.astype(jnp.float32) * scale

    return pl.pallas_call(
        body,
        out_shape=jax.ShapeDtypeStruct((m_per, n), jnp.float32),
        in_specs=[
            pl.BlockSpec(memory_space=pltpu.VMEM),
            pl.BlockSpec(memory_space=pltpu.VMEM),
        ],
        out_specs=pl.BlockSpec(memory_space=pltpu.VMEM),
        scratch_shapes=[
            pltpu.VMEM((N_DEV - 1, m_per, k_shard), jnp.bfloat16),
            pltpu.VMEM((m_per, n), jnp.float32),
            pltpu.VMEM((8, 128), jnp.float32),
            pltpu.VMEM((N_DEV - 1, 8, 128), jnp.float32),
            pltpu.SemaphoreType.DMA((N_DEV - 1,)),
            pltpu.SemaphoreType.DMA((N_DEV - 1,)),
            pltpu.SemaphoreType.DMA((N_DEV - 1,)),
            pltpu.SemaphoreType.DMA((N_DEV - 1,)),
        ],
        compiler_params=pltpu.CompilerParams(
            collective_id=0,
            vmem_limit_bytes=100 * 1024 * 1024,
        ),
    )(x, w_mat)


# device time: 79705 ns/iter; 1.3935x vs baseline; 1.3935x over previous
import jax
import jax.numpy as jnp
from jax import lax
from jax.experimental import pallas as pl
from jax.experimental.pallas import tpu as pltpu

N_DEV = 4


def kernel(x, w_mat):
    m_glob, k_shard = x.shape
    k_glob, n = w_mat.shape
    m_per = m_glob // N_DEV
    n_half = n // 2

    def body(x_hbm, w_hbm, out_ref,
             xtmp, send_ref, recv_ref, wtmp, wb, acc_ref,
             amax_src, amax_recv,
             xdma_sems, wdma_sems, send_sems, recv_sems,
             a_send_sems, a_recv_sems):
        me = lax.axis_index("i")

        hops = (1, 3, 2)

        m_h = m_per // 2
        t1 = lax.rem(me + 1, N_DEV)
        t3 = lax.rem(me + 3, N_DEV)
        t2 = lax.rem(me + 2, N_DEV)
        flat = [(ss, t, half)
                for (ss, t) in ((0, t1), (2, t3), (1, t2), (3, me))
                for half in (0, 1)]

        def x_load(slot, t, half):
            return pltpu.make_async_copy(
                x_hbm.at[pl.ds(t * m_per + half * m_h, m_h), :],
                xtmp.at[slot],
                xdma_sems.at[slot],
            )

        x_load(0, flat[0][1], flat[0][2]).start()
        x_load(1, flat[1][1], flat[1][2]).start()

        barrier_sem = pltpu.get_barrier_semaphore()
        for h in hops:
            peer = lax.rem(me + h, N_DEV)
            pl.semaphore_signal(
                barrier_sem, inc=1,
                device_id=(peer,), device_id_type=pl.DeviceIdType.MESH,
            )
        pl.semaphore_wait(barrier_sem, N_DEV - 1)

        rdmas = {}
        for i, (ss, t, half) in enumerate(flat):
            slot = i % 2
            x_load(slot, t, half).wait()
            send_ref[ss, pl.ds(half * m_h, m_h), :] = (
                xtmp[slot].astype(jnp.bfloat16))
            if i + 2 < len(flat):
                ns, nt, nh = flat[i + 2]
                x_load(slot, nt, nh).start()
            if half == 1 and ss != 3:
                rdma = pltpu.make_async_remote_copy(
                    src_ref=send_ref.at[ss],
                    dst_ref=recv_ref.at[ss],
                    send_sem=send_sems.at[ss],
                    recv_sem=recv_sems.at[ss],
                    device_id=(t,),
                    device_id_type=pl.DeviceIdType.MESH,
                )
                rdma.start()
                rdmas[ss] = rdma

        n_q = n // 4
        s_own = me
        s1 = lax.rem(me + N_DEV - 1, N_DEV)
        s3 = lax.rem(me + N_DEV - 3, N_DEV)
        s2 = lax.rem(me + N_DEV - 2, N_DEV)
        blocks = [
            (send_ref.at[3], s_own, None),
            (recv_ref.at[0], s1, 0),
            (recv_ref.at[2], s3, 2),
            (recv_ref.at[1], s2, 1),
        ]
        chunks = [(b, s, ss, j) for (b, s, ss) in blocks for j in range(4)]

        def w_load(c):
            _, s, _, j = chunks[c]
            return pltpu.make_async_copy(
                w_hbm.at[pl.ds(s * k_shard, k_shard), pl.ds(j * n_q, n_q)],
                wtmp.at[c % 2],
                wdma_sems.at[c % 2],
            )

        w_load(0).start()
        for c, (buf, s, ss, j) in enumerate(chunks):
            w_load(c).wait()
            wb[c % 2] = wtmp[c % 2].astype(jnp.bfloat16)
            if c + 1 < len(chunks):
                w_load(c + 1).start()
            if ss is not None and j == 0:
                rdmas[ss].wait_recv()
            partial = jnp.dot(buf[...], wb[c % 2],
                              preferred_element_type=jnp.float32)
            if c < 4:
                acc_ref[:, pl.ds(j * n_q, n_q)] = partial
            else:
                acc_ref[:, pl.ds(j * n_q, n_q)] += partial

        for ss in (0, 1, 2):
            rdmas[ss].wait_send()

        la = None
        for p in range(4):
            sl = pl.ds(p * n_q, n_q)
            chunk = jnp.maximum(acc_ref[:, sl], 0.0)
            acc_ref[:, sl] = chunk
            m = jnp.max(chunk)
            la = m if la is None else jnp.maximum(la, m)
        amax_src[...] = jnp.full((8, 128), la, jnp.float32)

        a_rdmas = []
        for h in hops:
            t = lax.rem(me + h, N_DEV)
            r = pltpu.make_async_remote_copy(
                src_ref=amax_src,
                dst_ref=amax_recv.at[h - 1],
                send_sem=a_send_sems.at[h - 1],
                recv_sem=a_recv_sems.at[h - 1],
                device_id=(t,),
                device_id_type=pl.DeviceIdType.MESH,
            )
            r.start()
            a_rdmas.append(r)
        for r in a_rdmas:
            r.wait()

        g_amax = jnp.maximum(la, jnp.max(amax_recv[...]))
        scale = g_amax / 448.0
        inv = 1.0 / scale
        for p in range(4):
            sl = pl.ds(p * n_q, n_q)
            q = (acc_ref[:, sl] * inv).astype(jnp.float8_e4m3fn)
            out_ref[:, sl] = q.astype(jnp.float32) * scale

    return pl.pallas_call(
        body,
        out_shape=jax.ShapeDtypeStruct((m_per, n), jnp.float32),
        in_specs=[
            pl.BlockSpec(memory_space=pltpu.MemorySpace.HBM),
            pl.BlockSpec(memory_space=pltpu.MemorySpace.HBM),
        ],
        out_specs=pl.BlockSpec(memory_space=pltpu.VMEM),
        scratch_shapes=[
            pltpu.VMEM((2, m_per // 2, k_shard), jnp.float32),
            pltpu.VMEM((N_DEV, m_per, k_shard), jnp.bfloat16),
            pltpu.VMEM((N_DEV - 1, m_per, k_shard), jnp.bfloat16),
            pltpu.VMEM((2, k_shard, n // 4), jnp.float32),
            pltpu.VMEM((2, k_shard, n // 4), jnp.bfloat16),
            pltpu.VMEM((m_per, n), jnp.float32),
            pltpu.VMEM((8, 128), jnp.float32),
            pltpu.VMEM((N_DEV - 1, 8, 128), jnp.float32),
            pltpu.SemaphoreType.DMA((2,)),
            pltpu.SemaphoreType.DMA((2,)),
            pltpu.SemaphoreType.DMA((N_DEV - 1,)),
            pltpu.SemaphoreType.DMA((N_DEV - 1,)),
            pltpu.SemaphoreType.DMA((N_DEV - 1,)),
            pltpu.SemaphoreType.DMA((N_DEV - 1,)),
        ],
        compiler_params=pltpu.CompilerParams(
            collective_id=0,
            vmem_limit_bytes=60 * 1024 * 1024,
        ),
    )(x, w_mat)


# device time: 75515 ns/iter; 1.4708x vs baseline; 1.0555x over previous
import jax
import jax.numpy as jnp
from jax import lax
from jax.experimental import pallas as pl
from jax.experimental.pallas import tpu as pltpu

N_DEV = 4


def kernel(x, w_mat):
    m_glob, k_shard = x.shape
    k_glob, n = w_mat.shape
    m_per = m_glob // N_DEV
    n_half = n // 2

    def body(x_hbm, w_hbm, out_ref,
             xtmp, send_ref, recv_ref, wtmp, wb, wbd, acc_ref,
             amax_src, amax_recv,
             xdma_sems, wdma_sems, send_sems, recv_sems,
             a_send_sems, a_recv_sems):
        me = lax.axis_index("i")

        hops = (1, 3, 2)

        m_h = m_per // 2
        t1 = lax.rem(me + 1, N_DEV)
        t3 = lax.rem(me + 3, N_DEV)
        t2 = lax.rem(me + 2, N_DEV)
        flat = [(ss, t, half)
                for (ss, t) in ((0, t1), (2, t3), (1, t2), (3, me))
                for half in (0, 1)]

        def x_load(slot, t, half):
            return pltpu.make_async_copy(
                x_hbm.at[pl.ds(t * m_per + half * m_h, m_h), :],
                xtmp.at[slot],
                xdma_sems.at[slot],
            )

        x_load(0, flat[0][1], flat[0][2]).start()
        x_load(1, flat[1][1], flat[1][2]).start()

        barrier_sem = pltpu.get_barrier_semaphore()
        for h in hops:
            peer = lax.rem(me + h, N_DEV)
            pl.semaphore_signal(
                barrier_sem, inc=1,
                device_id=(peer,), device_id_type=pl.DeviceIdType.MESH,
            )
        pl.semaphore_wait(barrier_sem, N_DEV - 1)

        rdmas = {}
        for i, (ss, t, half) in enumerate(flat):
            slot = i % 2
            x_load(slot, t, half).wait()
            send_ref[ss, pl.ds(half * m_h, m_h), :] = (
                xtmp[slot].astype(jnp.bfloat16))
            if i + 2 < len(flat):
                ns, nt, nh = flat[i + 2]
                x_load(slot, nt, nh).start()
            if half == 1 and ss != 3:
                rdma = pltpu.make_async_remote_copy(
                    src_ref=send_ref.at[ss],
                    dst_ref=recv_ref.at[ss],
                    send_sem=send_sems.at[ss],
                    recv_sem=recv_sems.at[ss],
                    device_id=(t,),
                    device_id_type=pl.DeviceIdType.MESH,
                )
                rdma.start()
                rdmas[ss] = rdma

        n_q = n // 4
        s_own = me
        s1 = lax.rem(me + N_DEV - 1, N_DEV)
        s3 = lax.rem(me + N_DEV - 3, N_DEV)
        s2 = lax.rem(me + N_DEV - 2, N_DEV)
        blocks = [
            (send_ref.at[3], s_own, None),
            (recv_ref.at[0], s1, 0),
            (recv_ref.at[2], s3, 2),
        ]
        chunks = [(b, s, ss, j) for (b, s, ss) in blocks for j in range(4)]

        def w_chunk_load(s, j, slot):
            return pltpu.make_async_copy(
                w_hbm.at[pl.ds(s * k_shard, k_shard), pl.ds(j * n_q, n_q)],
                wtmp.at[slot],
                wdma_sems.at[slot],
            )

        w_chunk_load(s2, 0, 0).start()
        w_chunk_load(s2, 1, 1).start()
        for j in range(4):
            slot = j % 2
            w_chunk_load(s2, j, slot).wait()
            wbd[:, pl.ds(j * n_q, n_q)] = wtmp[slot].astype(jnp.bfloat16)
            if j + 2 < 4:
                w_chunk_load(s2, j + 2, slot).start()

        def w_load(c):
            _, s, _, j = chunks[c]
            return w_chunk_load(s, j, c % 2)

        w_load(0).start()
        for c, (buf, s, ss, j) in enumerate(chunks):
            w_load(c).wait()
            wb[c % 2] = wtmp[c % 2].astype(jnp.bfloat16)
            if c + 1 < len(chunks):
                w_load(c + 1).start()
            if ss is not None and j == 0:
                rdmas[ss].wait_recv()
            partial = jnp.dot(buf[...], wb[c % 2],
                              preferred_element_type=jnp.float32)
            if c < 4:
                acc_ref[:, pl.ds(j * n_q, n_q)] = partial
            else:
                acc_ref[:, pl.ds(j * n_q, n_q)] += partial

        rdmas[1].wait_recv()
        la = None
        for j in range(4):
            sl = pl.ds(j * n_q, n_q)
            quarter = jnp.maximum(
                acc_ref[:, sl] + jnp.dot(recv_ref[1], wbd[:, sl],
                                         preferred_element_type=jnp.float32),
                0.0)
            acc_ref[:, sl] = quarter
            m = jnp.max(quarter)
            la = m if la is None else jnp.maximum(la, m)

        for ss in (0, 1, 2):
            rdmas[ss].wait_send()
        amax_src[...] = jnp.full((8, 128), la, jnp.float32)

        a_rdmas = []
        for h in hops:
            t = lax.rem(me + h, N_DEV)
            r = pltpu.make_async_remote_copy(
                src_ref=amax_src,
                dst_ref=amax_recv.at[h - 1],
                send_sem=a_send_sems.at[h - 1],
                recv_sem=a_recv_sems.at[h - 1],
                device_id=(t,),
                device_id_type=pl.DeviceIdType.MESH,
            )
            r.start()
            a_rdmas.append(r)
        for r in a_rdmas:
            r.wait()

        g_amax = jnp.maximum(la, jnp.max(amax_recv[...]))
        scale = g_amax / 448.0
        inv = 1.0 / scale
        for p in range(4):
            sl = pl.ds(p * n_q, n_q)
            q = (acc_ref[:, sl] * inv).astype(jnp.float8_e4m3fn)
            out_ref[:, sl] = q.astype(jnp.float32) * scale

    return pl.pallas_call(
        body,
        out_shape=jax.ShapeDtypeStruct((m_per, n), jnp.float32),
        in_specs=[
            pl.BlockSpec(memory_space=pltpu.MemorySpace.HBM),
            pl.BlockSpec(memory_space=pltpu.MemorySpace.HBM),
        ],
        out_specs=pl.BlockSpec(memory_space=pltpu.VMEM),
        scratch_shapes=[
            pltpu.VMEM((2, m_per // 2, k_shard), jnp.float32),
            pltpu.VMEM((N_DEV, m_per, k_shard), jnp.bfloat16),
            pltpu.VMEM((N_DEV - 1, m_per, k_shard), jnp.bfloat16),
            pltpu.VMEM((2, k_shard, n // 4), jnp.float32),
            pltpu.VMEM((2, k_shard, n // 4), jnp.bfloat16),
            pltpu.VMEM((k_shard, n), jnp.bfloat16),
            pltpu.VMEM((m_per, n), jnp.float32),
            pltpu.VMEM((8, 128), jnp.float32),
            pltpu.VMEM((N_DEV - 1, 8, 128), jnp.float32),
            pltpu.SemaphoreType.DMA((2,)),
            pltpu.SemaphoreType.DMA((2,)),
            pltpu.SemaphoreType.DMA((N_DEV - 1,)),
            pltpu.SemaphoreType.DMA((N_DEV - 1,)),
            pltpu.SemaphoreType.DMA((N_DEV - 1,)),
            pltpu.SemaphoreType.DMA((N_DEV - 1,)),
        ],
        compiler_params=pltpu.CompilerParams(
            collective_id=0,
            vmem_limit_bytes=63 * 1024 * 1024 + 512 * 1024,
        ),
    )(x, w_mat)


# device time: 75483 ns/iter; 1.4714x vs baseline; 1.0004x over previous
import jax
import jax.numpy as jnp
from jax import lax
from jax.experimental import pallas as pl
from jax.experimental.pallas import tpu as pltpu

N_DEV = 4


def kernel(x, w_mat):
    m_glob, k_shard = x.shape
    k_glob, n = w_mat.shape
    m_per = m_glob // N_DEV
    n_half = n // 2

    def body(x_hbm, w_hbm, out_ref,
             xtmp, send_ref, recv_ref, wtmp, wb, wbd, acc_ref,
             amax_src, amax_recv,
             xdma_sems, wdma_sems, send_sems, recv_sems,
             a_send_sems, a_recv_sems):
        me = lax.axis_index("i")

        hops = (1, 3, 2)

        m_h = m_per // 2
        t1 = lax.rem(me + 1, N_DEV)
        t3 = lax.rem(me + 3, N_DEV)
        t2 = lax.rem(me + 2, N_DEV)
        flat = [(ss, t, half)
                for (ss, t) in ((0, t1), (2, t3), (1, t2), (3, me))
                for half in (0, 1)]

        def x_load(slot, t, half):
            return pltpu.make_async_copy(
                x_hbm.at[pl.ds(t * m_per + half * m_h, m_h), :],
                xtmp.at[slot],
                xdma_sems.at[slot],
            )

        x_load(0, flat[0][1], flat[0][2]).start()
        x_load(1, flat[1][1], flat[1][2]).start()

        barrier_sem = pltpu.get_barrier_semaphore()
        for h in hops:
            peer = lax.rem(me + h, N_DEV)
            pl.semaphore_signal(
                barrier_sem, inc=1,
                device_id=(peer,), device_id_type=pl.DeviceIdType.MESH,
            )
        pl.semaphore_wait(barrier_sem, N_DEV - 1)

        rdmas = {}
        for i, (ss, t, half) in enumerate(flat):
            slot = i % 2
            x_load(slot, t, half).wait()
            send_ref[ss, pl.ds(half * m_h, m_h), :] = (
                xtmp[slot].astype(jnp.bfloat16))
            if i + 2 < len(flat):
                ns, nt, nh = flat[i + 2]
                x_load(slot, nt, nh).start()
            if half == 1 and ss != 3:
                rdma = pltpu.make_async_remote_copy(
                    src_ref=send_ref.at[ss],
                    dst_ref=recv_ref.at[ss],
                    send_sem=send_sems.at[ss],
                    recv_sem=recv_sems.at[ss],
                    device_id=(t,),
                    device_id_type=pl.DeviceIdType.MESH,
                )
                rdma.start()
                rdmas[ss] = rdma

        n_q = n // 4
        s_own = me
        s1 = lax.rem(me + N_DEV - 1, N_DEV)
        s3 = lax.rem(me + N_DEV - 3, N_DEV)
        s2 = lax.rem(me + N_DEV - 2, N_DEV)
        blocks = [
            (send_ref.at[3], s_own, None),
            (recv_ref.at[0], s1, 0),
            (recv_ref.at[2], s3, 2),
        ]
        chunks = [(b, s, ss, j) for (b, s, ss) in blocks for j in range(4)]

        def w_chunk_load(s, j, slot):
            return pltpu.make_async_copy(
                w_hbm.at[pl.ds(s * k_shard, k_shard), pl.ds(j * n_q, n_q)],
                wtmp.at[slot],
                wdma_sems.at[slot],
            )

        w_chunk_load(s2, 0, 0).start()
        w_chunk_load(s2, 1, 1).start()
        for j in range(4):
            slot = j % 2
            w_chunk_load(s2, j, slot).wait()
            wbd[:, pl.ds(j * n_q, n_q)] = wtmp[slot].astype(jnp.bfloat16)
            if j + 2 < 4:
                w_chunk_load(s2, j + 2, slot).start()

        def w_load(c):
            _, s, _, j = chunks[c]
            return w_chunk_load(s, j, c % 2)

        w_load(0).start()
        for c, (buf, s, ss, j) in enumerate(chunks):
            w_load(c).wait()
            wb[c % 2] = wtmp[c % 2].astype(jnp.bfloat16)
            if c + 1 < len(chunks):
                w_load(c + 1).start()
            if ss is not None and j == 0:
                rdmas[ss].wait_recv()
            partial = jnp.dot(buf[...], wb[c % 2],
                              preferred_element_type=jnp.float32)
            if c < 4:
                acc_ref[:, pl.ds(j * n_q, n_q)] = partial
            else:
                acc_ref[:, pl.ds(j * n_q, n_q)] += partial

        rdmas[1].wait_recv()
        la = None
        for j in range(4):
            sl = pl.ds(j * n_q, n_q)
            quarter = jnp.maximum(
                acc_ref[:, sl] + jnp.dot(recv_ref[1], wbd[:, sl],
                                         preferred_element_type=jnp.float32),
                0.0)
            acc_ref[:, sl] = quarter
            m = jnp.max(quarter)
            la = m if la is None else jnp.maximum(la, m)

        for ss in (0, 1, 2):
            rdmas[ss].wait_send()
        amax_src[...] = jnp.full((8, 128), la, jnp.float32)

        a_rdmas = []
        for h in hops:
            t = lax.rem(me + h, N_DEV)
            r = pltpu.make_async_remote_copy(
                src_ref=amax_src,
                dst_ref=amax_recv.at[h - 1],
                send_sem=a_send_sems.at[h - 1],
                recv_sem=a_recv_sems.at[h - 1],
                device_id=(t,),
                device_id_type=pl.DeviceIdType.MESH,
            )
            r.start()
            a_rdmas.append(r)
        for r in a_rdmas:
            r.wait()

        g_amax = jnp.maximum(la, jnp.max(amax_recv[...]))
        scale = g_amax / 448.0
        inv = 1.0 / scale
        out_dmas = []
        for p in range(4):
            sl = pl.ds(p * n_q, n_q)
            slot = p % 2
            if p >= 2:
                out_dmas[p - 2].wait()
            q = (acc_ref[:, sl] * inv).astype(jnp.float8_e4m3fn)
            wtmp[slot] = q.astype(jnp.float32) * scale
            d = pltpu.make_async_copy(
                wtmp.at[slot], out_ref.at[:, sl], wdma_sems.at[slot])
            d.start()
            out_dmas.append(d)
        out_dmas[2].wait()
        out_dmas[3].wait()

    return pl.pallas_call(
        body,
        out_shape=jax.ShapeDtypeStruct((m_per, n), jnp.float32),
        in_specs=[
            pl.BlockSpec(memory_space=pltpu.MemorySpace.HBM),
            pl.BlockSpec(memory_space=pltpu.MemorySpace.HBM),
        ],
        out_specs=pl.BlockSpec(memory_space=pltpu.MemorySpace.HBM),
        scratch_shapes=[
            pltpu.VMEM((2, m_per // 2, k_shard), jnp.float32),
            pltpu.VMEM((N_DEV, m_per, k_shard), jnp.bfloat16),
            pltpu.VMEM((N_DEV - 1, m_per, k_shard), jnp.bfloat16),
            pltpu.VMEM((2, k_shard, n // 4), jnp.float32),
            pltpu.VMEM((2, k_shard, n // 4), jnp.bfloat16),
            pltpu.VMEM((k_shard, n), jnp.bfloat16),
            pltpu.VMEM((m_per, n), jnp.float32),
            pltpu.VMEM((8, 128), jnp.float32),
            pltpu.VMEM((N_DEV - 1, 8, 128), jnp.float32),
            pltpu.SemaphoreType.DMA((2,)),
            pltpu.SemaphoreType.DMA((2,)),
            pltpu.SemaphoreType.DMA((N_DEV - 1,)),
            pltpu.SemaphoreType.DMA((N_DEV - 1,)),
            pltpu.SemaphoreType.DMA((N_DEV - 1,)),
            pltpu.SemaphoreType.DMA((N_DEV - 1,)),
        ],
        compiler_params=pltpu.CompilerParams(
            collective_id=0,
            vmem_limit_bytes=63 * 1024 * 1024 + 512 * 1024,
        ),
    )(x, w_mat)


# device time: 72198 ns/iter; 1.5384x vs baseline; 1.0455x over previous
import jax
import jax.numpy as jnp
from jax import lax
from jax.experimental import pallas as pl
from jax.experimental.pallas import tpu as pltpu

N_DEV = 4


def kernel(x, w_mat):
    m_glob, k_shard = x.shape
    k_glob, n = w_mat.shape
    m_per = m_glob // N_DEV
    n_half = n // 2

    def body(x_hbm, w_hbm, out_ref,
             xtmp, send_ref, recv_ref, wtmp, wb, wbd, acc_ref,
             amax_src, amax_recv,
             xdma_sems, wdma_sems, send_sems, recv_sems,
             a_send_sems, a_recv_sems):
        me = lax.axis_index("i")

        hops = (1, 3, 2)

        m_h = m_per // 2
        t1 = lax.rem(me + 1, N_DEV)
        t3 = lax.rem(me + 3, N_DEV)
        t2 = lax.rem(me + 2, N_DEV)
        flat = [(ss, t, half)
                for (ss, t) in ((0, t1), (2, t3), (1, t2), (3, me))
                for half in (0, 1)]

        def x_load(slot, t, half):
            return pltpu.make_async_copy(
                x_hbm.at[pl.ds(t * m_per + half * m_h, m_h), :],
                xtmp.at[slot],
                xdma_sems.at[slot],
            )

        x_load(0, flat[0][1], flat[0][2]).start()
        x_load(1, flat[1][1], flat[1][2]).start()

        barrier_sem = pltpu.get_barrier_semaphore()
        for h in hops:
            peer = lax.rem(me + h, N_DEV)
            pl.semaphore_signal(
                barrier_sem, inc=1,
                device_id=(peer,), device_id_type=pl.DeviceIdType.MESH,
            )
        pl.semaphore_wait(barrier_sem, N_DEV - 1)

        rdmas = {}
        for i, (ss, t, half) in enumerate(flat):
            slot = i % 2
            x_load(slot, t, half).wait()
            send_ref[ss, pl.ds(half * m_h, m_h), :] = (
                xtmp[slot].astype(jnp.bfloat16))
            if i + 2 < len(flat):
                ns, nt, nh = flat[i + 2]
                x_load(slot, nt, nh).start()
            if half == 1 and ss != 3:
                rdma = pltpu.make_async_remote_copy(
                    src_ref=send_ref.at[ss],
                    dst_ref=recv_ref.at[ss],
                    send_sem=send_sems.at[ss],
                    recv_sem=recv_sems.at[ss],
                    device_id=(t,),
                    device_id_type=pl.DeviceIdType.MESH,
                )
                rdma.start()
                rdmas[ss] = rdma

        n_q = n // 4
        s_own = me
        s1 = lax.rem(me + N_DEV - 1, N_DEV)
        s3 = lax.rem(me + N_DEV - 3, N_DEV)
        s2 = lax.rem(me + N_DEV - 2, N_DEV)
        blocks = [
            (send_ref.at[3], s_own, None),
            (recv_ref.at[0], s1, 0),
            (recv_ref.at[2], s3, 2),
        ]
        chunks = [(b, s, ss, j) for (b, s, ss) in blocks for j in range(4)]

        def w_chunk_load(s, j, slot):
            return pltpu.make_async_copy(
                w_hbm.at[pl.ds(s * k_shard, k_shard), pl.ds(j * n_q, n_q)],
                wtmp.at[slot],
                wdma_sems.at[slot],
            )

        w_chunk_load(s2, 0, 0).start()
        w_chunk_load(s2, 1, 1).start()
        for j in range(4):
            slot = j % 2
            w_chunk_load(s2, j, slot).wait()
            wbd[:, pl.ds(j * n_q, n_q)] = wtmp[slot].astype(jnp.bfloat16)
            if j + 2 < 4:
                w_chunk_load(s2, j + 2, slot).start()

        def w_load(c):
            _, s, _, j = chunks[c]
            return w_chunk_load(s, j, c % 2)

        w_load(0).start()
        for c, (buf, s, ss, j) in enumerate(chunks):
            w_load(c).wait()
            wb[c % 2] = wtmp[c % 2].astype(jnp.bfloat16)
            if c + 1 < len(chunks):
                w_load(c + 1).start()
            if ss is not None and j == 0:
                rdmas[ss].wait_recv()
            partial = jnp.dot(buf[...], wb[c % 2],
                              preferred_element_type=jnp.float32)
            if c < 4:
                acc_ref[:, pl.ds(j * n_q, n_q)] = partial
            else:
                acc_ref[:, pl.ds(j * n_q, n_q)] += partial

        rdmas[1].wait_recv()
        la = None
        for j in range(4):
            sl = pl.ds(j * n_q, n_q)
            quarter = jnp.maximum(
                acc_ref[:, sl] + jnp.dot(recv_ref[1], wbd[:, sl],
                                         preferred_element_type=jnp.float32),
                0.0)
            acc_ref[:, sl] = quarter
            m = jnp.max(quarter)
            la = m if la is None else jnp.maximum(la, m)

        for ss in (0, 1, 2):
            rdmas[ss].wait_send()
        amax_src[...] = jnp.full((8, 128), la, jnp.float32)

        a_rdmas = []
        for h in hops:
            t = lax.rem(me + h, N_DEV)
            r = pltpu.make_async_remote_copy(
                src_ref=amax_src,
                dst_ref=amax_recv.at[h - 1],
                send_sem=a_send_sems.at[h - 1],
                recv_sem=a_recv_sems.at[h - 1],
                device_id=(t,),
                device_id_type=pl.DeviceIdType.MESH,
            )
            r.start()
            a_rdmas.append(r)
        for r in a_rdmas:
            r.wait()

        g_amax = jnp.maximum(la, jnp.max(amax_recv[...]))
        scale = g_amax / 448.0
        inv = 1.0 / scale
        out_dmas = []
        for p in range(4):
            sl = pl.ds(p * n_q, n_q)
            slot = p % 2
            if p >= 2:
                out_dmas[p - 2].wait()
            q = (acc_ref[:, sl] * inv).astype(jnp.float8_e4m3fn)
            wb[slot] = (q.astype(jnp.float32) * scale).astype(jnp.bfloat16)
            d = pltpu.make_async_copy(
                wb.at[slot], out_ref.at[:, sl], wdma_sems.at[slot])
            d.start()
            out_dmas.append(d)
        out_dmas[2].wait()
        out_dmas[3].wait()

    return pl.pallas_call(
        body,
        out_shape=jax.ShapeDtypeStruct((m_per, n), jnp.bfloat16),
        in_specs=[
            pl.BlockSpec(memory_space=pltpu.MemorySpace.HBM),
            pl.BlockSpec(memory_space=pltpu.MemorySpace.HBM),
        ],
        out_specs=pl.BlockSpec(memory_space=pltpu.MemorySpace.HBM),
        scratch_shapes=[
            pltpu.VMEM((2, m_per // 2, k_shard), jnp.float32),
            pltpu.VMEM((N_DEV, m_per, k_shard), jnp.bfloat16),
            pltpu.VMEM((N_DEV - 1, m_per, k_shard), jnp.bfloat16),
            pltpu.VMEM((2, k_shard, n // 4), jnp.float32),
            pltpu.VMEM((2, k_shard, n // 4), jnp.bfloat16),
            pltpu.VMEM((k_shard, n), jnp.bfloat16),
            pltpu.VMEM((m_per, n), jnp.float32),
            pltpu.VMEM((8, 128), jnp.float32),
            pltpu.VMEM((N_DEV - 1, 8, 128), jnp.float32),
            pltpu.SemaphoreType.DMA((2,)),
            pltpu.SemaphoreType.DMA((2,)),
            pltpu.SemaphoreType.DMA((N_DEV - 1,)),
            pltpu.SemaphoreType.DMA((N_DEV - 1,)),
            pltpu.SemaphoreType.DMA((N_DEV - 1,)),
            pltpu.SemaphoreType.DMA((N_DEV - 1,)),
        ],
        compiler_params=pltpu.CompilerParams(
            collective_id=0,
            vmem_limit_bytes=63 * 1024 * 1024 + 512 * 1024,
        ),
    )(x, w_mat)
